# baseline (speedup 1.0000x reference)
import functools

import jax
import jax.numpy as jnp
from jax.experimental import pallas as pl
from jax.experimental.pallas import tpu as pltpu

_NMS_THRESH = 0.2
_SCORE_THRESH = 0.5
_BIAS = 128
_ACTIVE_LIM = 8192 + _BIAS
_KEPT = 12288
_INACT = 16384
_NEG = -1e30
_CHUNK = 32


def _nms_body(n, rows, cols, x0_ref, y0_ref, x1_ref, y1_ref, area_ref, sc_ref,
              out_ref, ridx_ref, *key_refs):
    shape = (rows, cols)
    r = jax.lax.broadcasted_iota(jnp.int32, shape, 0)
    c = jax.lax.broadcasted_iota(jnp.int32, shape, 1)
    idx = r * cols + c
    valid = (sc_ref[:, :] > _SCORE_THRESH) & (idx < n)
    ridx_ref[:, :] = jnp.where(valid, idx + _BIAS, _INACT)

    for kref, (cref, half) in zip(
            key_refs,
            [(x0_ref, 1), (x0_ref, 0), (y0_ref, 1), (y0_ref, 0),
             (x1_ref, 1), (x1_ref, 0), (y1_ref, 1), (y1_ref, 0)]):
        bits = jax.lax.bitcast_convert_type(cref[:, :], jnp.int32)
        kref[:, :] = (bits >> 16) & 0xFFFF if half else bits & 0xFFFF

    def reduce8(state):
        sh = state << 16
        mins = []
        for kref in key_refs:
            key = jax.lax.bitcast_convert_type(sh | kref[:, :], jnp.float32)
            m = jnp.min(key, axis=1, keepdims=True)
            m = jnp.min(m, axis=0, keepdims=True)
            mins.append(jax.lax.bitcast_convert_type(m, jnp.int32))
        return tuple(mins)

    def unpack(ok, hi, lo):
        bits = ((hi & 0xFFFF) << 16) | (lo & 0xFFFF)
        coord = jax.lax.bitcast_convert_type(bits, jnp.float32)
        return jnp.where(ok, coord, _NEG)

    def one_step(_, carry):
        mins, gate_bad = carry
        idx_min = mins[0] >> 16
        ok = (idx_min < _ACTIVE_LIM) & (gate_bad == 0)
        xi0 = unpack(ok, mins[0], mins[1])
        yi0 = unpack(ok, mins[2], mins[3])
        xi1 = unpack(ok, mins[4], mins[5])
        yi1 = unpack(ok, mins[6], mins[7])
        ai = (xi1 - xi0) * (yi1 - yi0)
        ridx = ridx_ref[:, :]
        is_i = (ridx == idx_min) & ok
        nmins = reduce8(jnp.where(is_i, _KEPT, ridx))
        x0 = x0_ref[:, :]
        y0 = y0_ref[:, :]
        x1 = x1_ref[:, :]
        y1 = y1_ref[:, :]
        w = jnp.maximum(jnp.minimum(xi1, x1) - jnp.maximum(xi0, x0), 0.0)
        h = jnp.maximum(jnp.minimum(yi1, y1) - jnp.maximum(yi0, y0), 0.0)
        inter = w * h
        sup = inter > _NMS_THRESH * (ai + area_ref[:, :] - inter)
        ridx_ref[:, :] = jnp.where(is_i, _KEPT, jnp.where(sup, _INACT, ridx))
        c_ok = (nmins[0] >> 16) < _ACTIVE_LIM
        cx0 = unpack(c_ok, nmins[0], nmins[1])
        cy0 = unpack(c_ok, nmins[2], nmins[3])
        cx1 = unpack(c_ok, nmins[4], nmins[5])
        cy1 = unpack(c_ok, nmins[6], nmins[7])
        ca = (cx1 - cx0) * (cy1 - cy0)
        wc = jnp.maximum(jnp.minimum(xi1, cx1) - jnp.maximum(xi0, cx0), 0.0)
        hc = jnp.maximum(jnp.minimum(yi1, cy1) - jnp.maximum(yi0, cy0), 0.0)
        ic = wc * hc
        gate = (ic > _NMS_THRESH * (ai + ca - ic)).astype(jnp.int32)
        return (nmins, gate)

    def chunk_cond(mn):
        return mn < _ACTIVE_LIM

    def chunk_body(mn):
        init = (reduce8(ridx_ref[:, :]), jnp.zeros((1, 1), jnp.int32))
        last, _ = jax.lax.fori_loop(0, _CHUNK, one_step, init, unroll=True)
        return jnp.min(last[0] >> 16)

    jax.lax.while_loop(chunk_cond, chunk_body, jnp.int32(0))
    out_ref[:, :] = jnp.where(ridx_ref[:, :] == _KEPT, sc_ref[:, :], 0.0)


def kernel(boxes, scores):
    n = scores.shape[0]
    rows, cols = 8, 640
    np_ = rows * cols
    _, x0s, y0s, x1s, y1s, ss = jax.lax.sort(
        (-scores, boxes[:, 0], boxes[:, 1], boxes[:, 2], boxes[:, 3], scores),
        num_keys=1)
    area = (x1s - x0s) * (y1s - y0s)

    def grid2d(v):
        return jnp.pad(v, (0, np_ - n)).reshape(rows, cols)

    planes = [grid2d(x0s), grid2d(y0s), grid2d(x1s), grid2d(y1s),
              grid2d(area), grid2d(ss)]

    out = pl.pallas_call(
        functools.partial(_nms_body, n, rows, cols),
        out_shape=jax.ShapeDtypeStruct((rows, cols), jnp.float32),
        scratch_shapes=[pltpu.VMEM((rows, cols), jnp.int32)] * 9,
    )(*planes)
    return out.reshape(np_)[:n]


# speedup vs baseline: 1.0444x; 1.0444x over previous
import functools

import jax
import jax.numpy as jnp
from jax.experimental import pallas as pl
from jax.experimental.pallas import tpu as pltpu

_NMS_THRESH = 0.2
_SCORE_THRESH = 0.5
_BIAS = 128
_ACTIVE_LIM = 8192 + _BIAS
_KEPT = 12288
_INACT = 16384
_NEG = -1e30
_CHUNK = 32


def _nms_body(n, rows, cols, x0_ref, y0_ref, x1_ref, y1_ref, area_ref, sc_ref,
              out_ref, ridx_ref, *key_refs):
    shape = (rows, cols)
    r = jax.lax.broadcasted_iota(jnp.int32, shape, 0)
    c = jax.lax.broadcasted_iota(jnp.int32, shape, 1)
    idx = r * cols + c
    valid = (sc_ref[:, :] > _SCORE_THRESH) & (idx < n)
    ridx_ref[:, :] = jnp.where(valid, idx + _BIAS, _INACT)

    for kref, (cref, half) in zip(
            key_refs,
            [(x0_ref, 1), (x0_ref, 0), (y0_ref, 1), (y0_ref, 0),
             (x1_ref, 1), (x1_ref, 0), (y1_ref, 1), (y1_ref, 0)]):
        bits = jax.lax.bitcast_convert_type(cref[:, :], jnp.int32)
        kref[:, :] = (bits >> 16) & 0xFFFF if half else bits & 0xFFFF

    def one_step(_, carry):
        ridx = ridx_ref[:, :]
        sh = ridx << 16
        mins = []
        for kref in key_refs:
            key = jax.lax.bitcast_convert_type(sh | kref[:, :], jnp.float32)
            m = jnp.min(key, axis=1, keepdims=True)
            m = jnp.min(m, axis=0, keepdims=True)
            mins.append(jax.lax.bitcast_convert_type(m, jnp.int32))
        idx_min = mins[0] >> 16
        ok = idx_min < _ACTIVE_LIM

        def unpack(hi, lo):
            bits = ((hi & 0xFFFF) << 16) | (lo & 0xFFFF)
            coord = jax.lax.bitcast_convert_type(bits, jnp.float32)
            return jnp.where(ok, coord, _NEG)

        xi0 = unpack(mins[0], mins[1])
        yi0 = unpack(mins[2], mins[3])
        xi1 = unpack(mins[4], mins[5])
        yi1 = unpack(mins[6], mins[7])
        ai = (xi1 - xi0) * (yi1 - yi0)
        x0 = x0_ref[:, :]
        y0 = y0_ref[:, :]
        x1 = x1_ref[:, :]
        y1 = y1_ref[:, :]
        w = jnp.maximum(jnp.minimum(xi1, x1) - jnp.maximum(xi0, x0), 0.0)
        h = jnp.maximum(jnp.minimum(yi1, y1) - jnp.maximum(yi0, y0), 0.0)
        inter = w * h
        sup = inter > _NMS_THRESH * (ai + area_ref[:, :] - inter)
        is_i = (ridx == idx_min) & ok
        ridx_ref[:, :] = jnp.where(is_i, _KEPT, jnp.where(sup, _INACT, ridx))
        return idx_min

    def chunk_cond(mn):
        return mn < _ACTIVE_LIM

    def chunk_body(mn):
        last = jax.lax.fori_loop(0, _CHUNK, one_step,
                                 jnp.zeros((1, 1), jnp.int32), unroll=True)
        return jnp.min(last)

    jax.lax.while_loop(chunk_cond, chunk_body, jnp.int32(0))
    out_ref[:, :] = jnp.where(ridx_ref[:, :] == _KEPT, sc_ref[:, :], 0.0)


def kernel(boxes, scores):
    n = scores.shape[0]
    rows, cols = 8, 640
    np_ = rows * cols
    _, x0s, y0s, x1s, y1s, ss = jax.lax.sort(
        (-scores, boxes[:, 0], boxes[:, 1], boxes[:, 2], boxes[:, 3], scores),
        num_keys=1)
    area = (x1s - x0s) * (y1s - y0s)

    def grid2d(v):
        return jnp.pad(v, (0, np_ - n)).reshape(rows, cols)

    planes = [grid2d(x0s), grid2d(y0s), grid2d(x1s), grid2d(y1s),
              grid2d(area), grid2d(ss)]

    out = pl.pallas_call(
        functools.partial(_nms_body, n, rows, cols),
        out_shape=jax.ShapeDtypeStruct((rows, cols), jnp.float32),
        scratch_shapes=[pltpu.VMEM((rows, cols), jnp.int32)] * 9,
    )(*planes)
    return out.reshape(np_)[:n]
